# Initial kernel scaffold; baseline (speedup 1.0000x reference)
import functools

import jax
import jax.numpy as jnp
from jax import lax
from jax.experimental import pallas as pl
from jax.experimental.pallas import tpu as pltpu
from jax.experimental.pallas import tpu_sc as plsc

TAGSET = 50
PAD = 50
B, L = 4096, 200
N_TOK = B * L
NW = 32
TOK_W = N_TOK // NW
CH = 128
ROWS = TOK_W // CH
LENS_W = B // NW
LANES = 16


def _sc_body(yp_hbm, yt_hbm, lens_hbm, ce_out, nb_out,
             yt_v, idx_v, g_v, lens_v, st_f, st_i, sem):
    wid = lax.axis_index("s") * 2 + lax.axis_index("c")
    base = wid * TOK_W

    pltpu.sync_copy(yt_hbm.at[pl.ds(wid * ROWS, ROWS)], yt_v)
    pltpu.sync_copy(lens_hbm.at[pl.ds(wid * LENS_W, LENS_W)], lens_v)

    iota = lax.broadcasted_iota(jnp.int32, (LANES,), 0)

    def idx_row(r, _):
        tok0 = base + r * CH
        for k in range(CH // LANES):
            yt = yt_v[r, pl.ds(k * LANES, LANES)]
            lbl = jnp.minimum(yt, TAGSET - 1)
            idx_v[r, pl.ds(k * LANES, LANES)] = (
                (tok0 + k * LANES + iota) * TAGSET + lbl)
        return 0

    lax.fori_loop(0, ROWS, idx_row, 0)

    pltpu.async_copy(yp_hbm.at[idx_v], g_v, sem).wait()

    def acc_row(r, acc):
        for k in range(CH // LANES):
            yt = yt_v[r, pl.ds(k * LANES, LANES)]
            g = g_v[r, pl.ds(k * LANES, LANES)]
            acc = acc + jnp.where(yt < PAD, g, 0.0)
        return acc

    acc = lax.fori_loop(0, ROWS, acc_row, jnp.zeros((LANES,), jnp.float32))

    nb = jnp.zeros((LANES,), jnp.int32)
    for k in range(LENS_W // LANES):
        nb = nb + lens_v[pl.ds(k * LANES, LANES)]

    st_f[...] = acc
    st_i[...] = nb
    pltpu.sync_copy(st_f, ce_out.at[wid])
    pltpu.sync_copy(st_i, nb_out.at[wid])


@jax.jit
def _sc_call(yp, yt, ln):
    mesh = plsc.VectorSubcoreMesh(core_axis_name="c", subcore_axis_name="s")
    f = pl.kernel(
        _sc_body,
        out_type=[
            jax.ShapeDtypeStruct((NW, LANES), jnp.float32),
            jax.ShapeDtypeStruct((NW, LANES), jnp.int32),
        ],
        mesh=mesh,
        scratch_types=[
            pltpu.VMEM((ROWS, CH), jnp.int32),
            pltpu.VMEM((ROWS, CH), jnp.int32),
            pltpu.VMEM((ROWS, CH), jnp.float32),
            pltpu.VMEM((LENS_W,), jnp.int32),
            pltpu.VMEM((LANES,), jnp.float32),
            pltpu.VMEM((LANES,), jnp.int32),
            pltpu.SemaphoreType.DMA,
        ],
    )
    return f(yp, yt, ln)


def kernel(y_pred, y_true, lens, masks):
    yp = y_pred.reshape(-1)
    yt = y_true.reshape(NW * ROWS, CH).astype(jnp.int32)
    ln = lens.astype(jnp.int32)
    ce_p, nb_p = _sc_call(yp, yt, ln)
    return -jnp.sum(ce_p) / jnp.sum(nb_p).astype(jnp.float32)


# speedup vs baseline: 1.0059x; 1.0059x over previous
import jax
import jax.numpy as jnp
from jax import lax
from jax.experimental import pallas as pl
from jax.experimental.pallas import tpu as pltpu
from jax.experimental.pallas import tpu_sc as plsc

TAGSET = 50
PAD = 50
B, L = 4096, 200
N_TOK = B * L
NW = 32
TOK_W = N_TOK // NW
LENS_W = B // NW
LANES = 16
UNROLL = 8
ROWS = TOK_W // (LANES * UNROLL)


def _sc_body(yp_hbm, yt_hbm, lens_hbm, ce_out, nb_out,
             yt_v, idx_v, g_v, lens_v, st_f, st_i, sem):
    wid = lax.axis_index("s") * 2 + lax.axis_index("c")
    base = wid * TOK_W

    pltpu.sync_copy(yt_hbm.at[pl.ds(base, TOK_W)], yt_v)
    pltpu.sync_copy(lens_hbm.at[pl.ds(wid * LENS_W, LENS_W)], lens_v)

    iota = lax.broadcasted_iota(jnp.int32, (LANES,), 0)

    def idx_row(r, _):
        off = r * (LANES * UNROLL)
        for k in range(UNROLL):
            yt = yt_v[pl.ds(off + k * LANES, LANES)]
            lbl = jnp.minimum(yt, TAGSET - 1)
            idx_v[pl.ds(off + k * LANES, LANES)] = (
                (base + off + k * LANES + iota) * TAGSET + lbl)
        return 0

    lax.fori_loop(0, ROWS, idx_row, 0)

    pltpu.async_copy(yp_hbm.at[idx_v], g_v, sem).wait()

    def acc_row(r, acc):
        off = r * (LANES * UNROLL)
        for k in range(UNROLL):
            yt = yt_v[pl.ds(off + k * LANES, LANES)]
            g = g_v[pl.ds(off + k * LANES, LANES)]
            acc = acc + jnp.where(yt < PAD, g, 0.0)
        return acc

    acc = lax.fori_loop(0, ROWS, acc_row, jnp.zeros((LANES,), jnp.float32))

    nb = jnp.zeros((LANES,), jnp.int32)
    for k in range(LENS_W // LANES):
        nb = nb + lens_v[pl.ds(k * LANES, LANES)]

    st_f[...] = acc
    st_i[...] = nb
    pltpu.sync_copy(st_f, ce_out.at[wid])
    pltpu.sync_copy(st_i, nb_out.at[wid])


@jax.jit
def _sc_call(yp, yt, ln):
    mesh = plsc.VectorSubcoreMesh(core_axis_name="c", subcore_axis_name="s")
    f = pl.kernel(
        _sc_body,
        out_type=[
            jax.ShapeDtypeStruct((NW, LANES), jnp.float32),
            jax.ShapeDtypeStruct((NW, LANES), jnp.int32),
        ],
        mesh=mesh,
        scratch_types=[
            pltpu.VMEM((TOK_W,), jnp.int32),
            pltpu.VMEM((TOK_W,), jnp.int32),
            pltpu.VMEM((TOK_W,), jnp.float32),
            pltpu.VMEM((LENS_W,), jnp.int32),
            pltpu.VMEM((LANES,), jnp.float32),
            pltpu.VMEM((LANES,), jnp.int32),
            pltpu.SemaphoreType.DMA,
        ],
    )
    return f(yp, yt, ln)


def kernel(y_pred, y_true, lens, masks):
    yp = y_pred.reshape(-1)
    yt = y_true.reshape(-1).astype(jnp.int32)
    ln = lens.astype(jnp.int32)
    ce_p, nb_p = _sc_call(yp, yt, ln)
    return -jnp.sum(ce_p) / jnp.sum(nb_p).astype(jnp.float32)
